# baseline (speedup 1.0000x reference)
import functools

import jax
import jax.numpy as jnp
from jax import lax
from jax.experimental import pallas as pl
from jax.experimental.pallas import tpu as pltpu
from jax.experimental.pallas import tpu_sc as plsc

F32 = jnp.float32

N_NODES = 50000
NPAD = 51200
TROWS = NPAD // 16
N_EDGES = 800000
EPAD = 16 * 392 * 128
DUMMY = N_NODES
B = 16384
USER_OFFSET = 25000

_mesh = plsc.VectorSubcoreMesh(core_axis_name="c", subcore_axis_name="s")
_sc_params = pltpu.CompilerParams(use_tc_tiling_on_sc=False)


def _make_deg():
    @functools.partial(
        pl.kernel,
        out_type=jax.ShapeDtypeStruct((2, NPAD, 16), F32),
        mesh=_mesh,
        compiler_params=_sc_params,
        scratch_types=[
            pltpu.VMEM((196, 128), jnp.int32),
            pltpu.VMEM((128, 16), F32),
            pltpu.VMEM((128, 16), F32),
            pltpu.VMEM_SHARED((NPAD, 16), F32),
        ],
    )
    def deg_k(dstr, ones_hbm, zeros_hbm, out, dst_v, ones_v, zeros_v, acc):
        cid = lax.axis_index("c")
        sid = lax.axis_index("s")
        wid = cid * 16 + sid
        pltpu.sync_copy(dstr.at[wid], dst_v)
        pltpu.sync_copy(ones_hbm, ones_v)
        pltpu.sync_copy(zeros_hbm, zeros_v)
        for z in range(TROWS // 128):
            pltpu.sync_copy(zeros_v, acc.at[pl.ds(sid * TROWS + z * 128, 128)])
        plsc.subcore_barrier()

        def body(j, carry):
            pltpu.sync_copy(ones_v, acc.at[dst_v.at[j]], add=True)
            return carry

        lax.fori_loop(0, 196, body, 0)
        plsc.subcore_barrier()
        pltpu.sync_copy(acc.at[pl.ds(sid * TROWS, TROWS)],
                        out.at[cid].at[pl.ds(sid * TROWS, TROWS)])

    return deg_k


def _make_agg(nc):
    @functools.partial(
        pl.kernel,
        out_type=jax.ShapeDtypeStruct((nc, NPAD, 32), F32),
        mesh=_mesh,
        compiler_params=_sc_params,
        scratch_types=[
            pltpu.VMEM((14, 128), jnp.int32),
            pltpu.VMEM((14, 128), jnp.int32),
            pltpu.VMEM((4, 128, 32), F32),
            pltpu.VMEM_SHARED((NPAD, 32), F32),
            pltpu.SemaphoreType.DMA,
            pltpu.SemaphoreType.DMA,
            pltpu.SemaphoreType.DMA,
            pltpu.SemaphoreType.DMA,
            pltpu.SemaphoreType.DMA,
            pltpu.SemaphoreType.DMA,
            pltpu.SemaphoreType.DMA,
            pltpu.SemaphoreType.DMA,
        ],
    )
    def agg_k(table, srcr, dstr, out,
              src_v, dst_v, rows_v, acc,
              sg0, sg1, sg2, sg3, ss0, ss1, ss2, ss3):
        cid = lax.axis_index("c")
        sid = lax.axis_index("s")
        sg = [sg0, sg1, sg2, sg3]
        ss = [ss0, ss1, ss2, ss3]
        for c in range(nc):
            @pl.when(cid == (c % 2))
            def _():
                pltpu.sync_copy(table.at[c].at[pl.ds(sid * TROWS, TROWS)],
                                acc.at[pl.ds(sid * TROWS, TROWS)])
                plsc.subcore_barrier()

                def body(g, carry):
                    base = g * 14
                    pltpu.sync_copy(srcr.at[sid].at[pl.ds(base, 14)], src_v)
                    pltpu.sync_copy(dstr.at[sid].at[pl.ds(base, 14)], dst_v)
                    gath = [None] * 4
                    for w in range(4):
                        gath[w] = pltpu.async_copy(
                            table.at[c].at[src_v.at[w]], rows_v.at[w], sg[w])
                    tail = []
                    for j in range(14):
                        w = j % 4
                        gath[w].wait()
                        sdesc = pltpu.async_copy(
                            rows_v.at[w], acc.at[dst_v.at[j]], ss[w],
                            add=True)
                        if j + 4 < 14:
                            sdesc.wait()
                            gath[w] = pltpu.async_copy(
                                table.at[c].at[src_v.at[j + 4]],
                                rows_v.at[w], sg[w])
                        else:
                            tail.append(sdesc)
                    for sdesc in tail:
                        sdesc.wait()
                    return carry

                lax.fori_loop(0, 392 // 14, body, 0)
                plsc.subcore_barrier()
                pltpu.sync_copy(acc.at[pl.ds(sid * TROWS, TROWS)],
                                out.at[c].at[pl.ds(sid * TROWS, TROWS)])

    return agg_k


def _make_gather():
    @functools.partial(
        pl.kernel,
        out_type=[
            jax.ShapeDtypeStruct((2, B, 32), F32),
            jax.ShapeDtypeStruct((2, B, 32), F32),
            jax.ShapeDtypeStruct((B, 32), F32),
            jax.ShapeDtypeStruct((B, 32), F32),
        ],
        mesh=_mesh,
        compiler_params=_sc_params,
        scratch_types=[
            pltpu.VMEM((8, 128), jnp.int32),
            pltpu.VMEM((128, 32), F32),
            pltpu.VMEM((128, 32), F32),
            pltpu.VMEM((128, 32), F32),
            pltpu.SemaphoreType.DMA,
        ],
    )
    def gather_k(agg2, dis32, idxr, gu, gp, du, dp,
                 idx_v, r0_v, r1_v, rd_v, sem):
        cid = lax.axis_index("c")
        sid = lax.axis_index("s")
        wid = cid * 16 + sid

        def run(out, outd, base0):
            pltpu.sync_copy(idxr.at[wid], idx_v)
            for j in range(8):
                row = idx_v.at[j]
                base = base0 + j * 128
                pltpu.async_copy(agg2.at[0].at[row], r0_v, sem).wait()
                pltpu.sync_copy(r0_v, out.at[0].at[pl.ds(base, 128)])
                pltpu.async_copy(agg2.at[1].at[row], r1_v, sem).wait()
                pltpu.sync_copy(r1_v, out.at[1].at[pl.ds(base, 128)])
                pltpu.async_copy(dis32.at[row], rd_v, sem).wait()
                pltpu.sync_copy(rd_v, outd.at[pl.ds(base, 128)])

        @pl.when(cid == 0)
        def _():
            run(gu, du, sid * 1024)

        @pl.when(cid == 1)
        def _():
            run(gp, dp, sid * 1024)

    return gather_k


_deg_k = _make_deg()
_agg2_k = _make_agg(2)
_agg4_k = _make_agg(4)
_gather_k = _make_gather()


_BLK = 1600
_NBLK = NPAD // _BLK


def _prep_body(deg2_ref, x_ref, dis_ref, dis32_ref, u0_ref):
    deg = deg2_ref[0, :, 0:1] + deg2_ref[1, :, 0:1] + 1.0
    dis = lax.rsqrt(deg)
    dis_ref[...] = dis
    dis32_ref[...] = jnp.broadcast_to(dis, dis32_ref.shape)
    u0_ref[...] = x_ref[...] * dis


def _tc_prep(deg2, xp):
    return pl.pallas_call(
        _prep_body,
        grid=(_NBLK,),
        in_specs=[
            pl.BlockSpec((2, _BLK, 16), lambda i: (0, i, 0)),
            pl.BlockSpec((_BLK, 64), lambda i: (i, 0)),
        ],
        out_specs=[
            pl.BlockSpec((_BLK, 1), lambda i: (i, 0)),
            pl.BlockSpec((_BLK, 32), lambda i: (i, 0)),
            pl.BlockSpec((_BLK, 64), lambda i: (i, 0)),
        ],
        out_shape=[
            jax.ShapeDtypeStruct((NPAD, 1), F32),
            jax.ShapeDtypeStruct((NPAD, 32), F32),
            jax.ShapeDtypeStruct((NPAD, 64), F32),
        ],
    )(deg2, xp)


def _make_fused(nc_in, nc_out, with_w3):
    din = 32 * nc_in

    def body(agg_ref, dis_ref, w_ref, b_ref, g_ref, bt_ref, w3_ref,
             u_ref, y_ref, st_ref):
        p = pl.program_id(0)
        i = pl.program_id(1)
        dis = dis_ref[...]

        @pl.when(p == 0)
        def _():
            m = agg_ref[...] * dis
            y = jnp.dot(m, w_ref[...], preferred_element_type=F32) \
                + b_ref[...]
            y_ref[pl.ds(i * _BLK, _BLK), :] = y

            @pl.when(i == 0)
            def _():
                st_ref[...] = jnp.zeros_like(st_ref)

            rid = i * _BLK + lax.broadcasted_iota(jnp.int32, (_BLK, 1), 0)
            ym = jnp.where(rid < N_NODES, y, 0.0)
            st_ref[0:1] += jnp.sum(ym, axis=0, keepdims=True)
            st_ref[1:2] += jnp.sum(ym * ym, axis=0, keepdims=True)

        @pl.when(p == 1)
        def _():
            mean = st_ref[0:1] / float(N_NODES)
            var = st_ref[1:2] / float(N_NODES) - mean * mean
            inv = lax.rsqrt(var + 1e-5)
            y = y_ref[pl.ds(i * _BLK, _BLK), :]
            h = jnp.maximum((y - mean) * inv * g_ref[...] + bt_ref[...], 0.0)
            if with_w3:
                h = jnp.dot(h, w3_ref[...], preferred_element_type=F32)
            u_ref[...] = h * dis

    def run(agg, dis, w, b, g, bt, w3):
        dout = w.shape[1]
        return pl.pallas_call(
            body,
            grid=(2, _NBLK),
            in_specs=[
                pl.BlockSpec((_BLK, din),
                             lambda p, i: (jnp.where(p == 0, i, 0), 0)),
                pl.BlockSpec((_BLK, 1), lambda p, i: (i, 0)),
                pl.BlockSpec((din, dout), lambda p, i: (0, 0)),
                pl.BlockSpec((1, dout), lambda p, i: (0, 0)),
                pl.BlockSpec((1, dout), lambda p, i: (0, 0)),
                pl.BlockSpec((1, dout), lambda p, i: (0, 0)),
                pl.BlockSpec(w3.shape, lambda p, i: (0, 0)),
            ],
            out_specs=pl.BlockSpec((_BLK, 32 * nc_out),
                                   lambda p, i: (i, 0)),
            out_shape=jax.ShapeDtypeStruct((NPAD, 32 * nc_out), F32),
            scratch_shapes=[
                pltpu.VMEM((NPAD, dout), F32),
                pltpu.VMEM((8, dout), F32),
            ],
            compiler_params=pltpu.CompilerParams(
                vmem_limit_bytes=50 * 1024 * 1024),
        )(agg, dis, w, b, g, bt, w3)

    return run


_fused1 = _make_fused(2, 4, False)
_fused2 = _make_fused(4, 2, True)


def _bn_full(a, g, bt):
    m = jnp.mean(a, axis=0, keepdims=True)
    v = jnp.mean(a * a, axis=0, keepdims=True) - m * m
    return jnp.maximum((a - m) * lax.rsqrt(v + 1e-5) * g + bt, 0.0)


_PBLK = 4096
_PNB = B // _PBLK


def _pair_body(gu_ref, gp_ref, du_ref, dp_ref, b3_ref,
               p1_ref, pb1_ref, pg1_ref, pbt1_ref,
               p2_ref, pb2_ref, pg2_ref, pbt2_ref, p3_ref, pb3_ref,
               out_ref, a1_ref, a2_ref, st1_ref, st2_ref):
    p = pl.program_id(0)
    i = pl.program_id(1)
    rows = pl.ds(i * _PBLK, _PBLK)

    @pl.when(p == 0)
    def _():
        ue = gu_ref[...]
        pe = gp_ref[...]
        p1t = p1_ref[0:64]
        p1b = p1_ref[64:128]
        bias = (jnp.dot(b3_ref[...], p1t + p1b, preferred_element_type=F32)
                + pb1_ref[...])
        a1 = (du_ref[:, 0:1] * jnp.dot(ue, p1t, preferred_element_type=F32)
              + dp_ref[:, 0:1] * jnp.dot(pe, p1b, preferred_element_type=F32)
              + bias)
        a1_ref[rows, :] = a1

        @pl.when(i == 0)
        def _():
            st1_ref[...] = jnp.zeros_like(st1_ref)

        st1_ref[0:1] += jnp.sum(a1, axis=0, keepdims=True)
        st1_ref[1:2] += jnp.sum(a1 * a1, axis=0, keepdims=True)

    @pl.when(p == 1)
    def _():
        mean = st1_ref[0:1] / float(B)
        var = st1_ref[1:2] / float(B) - mean * mean
        z1 = jnp.maximum((a1_ref[rows, :] - mean) * lax.rsqrt(var + 1e-5)
                         * pg1_ref[...] + pbt1_ref[...], 0.0)
        a2 = jnp.dot(z1, p2_ref[...], preferred_element_type=F32) \
            + pb2_ref[...]
        a2_ref[rows, :] = a2

        @pl.when(i == 0)
        def _():
            st2_ref[...] = jnp.zeros_like(st2_ref)

        st2_ref[0:1] += jnp.sum(a2, axis=0, keepdims=True)
        st2_ref[1:2] += jnp.sum(a2 * a2, axis=0, keepdims=True)

    @pl.when(p == 2)
    def _():
        mean = st2_ref[0:1] / float(B)
        var = st2_ref[1:2] / float(B) - mean * mean
        z2 = jnp.maximum((a2_ref[rows, :] - mean) * lax.rsqrt(var + 1e-5)
                         * pg2_ref[...] + pbt2_ref[...], 0.0)
        a3 = jnp.dot(z2, p3_ref[...], preferred_element_type=F32) \
            + pb3_ref[...]
        out_ref[...] = jax.nn.sigmoid(a3) * 4.0 + 1.0


def _tc_pair(gu, gp, du, dp, b3, p1, pb1, pg1, pbt1,
             p2, pb2, pg2, pbt2, p3, pb3):
    full = lambda shape: pl.BlockSpec(shape, lambda p, i: tuple(
        0 for _ in shape))
    return pl.pallas_call(
        _pair_body,
        grid=(3, _PNB),
        in_specs=[
            pl.BlockSpec((_PBLK, 64), lambda p, i: (jnp.where(p == 0, i, 0), 0)),
            pl.BlockSpec((_PBLK, 64), lambda p, i: (jnp.where(p == 0, i, 0), 0)),
            pl.BlockSpec((_PBLK, 32), lambda p, i: (jnp.where(p == 0, i, 0), 0)),
            pl.BlockSpec((_PBLK, 32), lambda p, i: (jnp.where(p == 0, i, 0), 0)),
            full((1, 64)),
            full((128, 128)),
            full((1, 128)),
            full((1, 128)),
            full((1, 128)),
            full((128, 64)),
            full((1, 64)),
            full((1, 64)),
            full((1, 64)),
            full((64, 1)),
            full((1, 1)),
        ],
        out_specs=pl.BlockSpec((_PBLK, 1), lambda p, i: (i, 0)),
        out_shape=jax.ShapeDtypeStruct((B, 1), F32),
        scratch_shapes=[
            pltpu.VMEM((B, 128), F32),
            pltpu.VMEM((B, 64), F32),
            pltpu.VMEM((8, 128), F32),
            pltpu.VMEM((8, 64), F32),
        ],
        compiler_params=pltpu.CompilerParams(
            vmem_limit_bytes=50 * 1024 * 1024),
    )(gu, gp, du, dp, b3, p1, pb1, pg1, pbt1, p2, pb2, pg2, pbt2, p3, pb3)


def kernel(x, edge_index, user_indices, product_indices,
           W1, b1, g1, bt1, W2, b2, g2, bt2, W3, b3,
           P1, pb1, pg1, pbt1, P2, pb2, pg2, pbt2, P3, pb3):
    src = jnp.concatenate(
        [edge_index[0], jnp.zeros((EPAD - N_EDGES,), jnp.int32)])
    dst = jnp.concatenate(
        [edge_index[1], jnp.full((EPAD - N_EDGES,), DUMMY, jnp.int32)])
    srcr16 = src.reshape(16, 392, 128)
    dstr16 = dst.reshape(16, 392, 128)
    dstr32 = dst.reshape(32, 196, 128)
    xp = jnp.pad(x, ((0, NPAD - N_NODES), (0, 0)))
    idxr = jnp.concatenate(
        [user_indices + USER_OFFSET, product_indices]).reshape(32, 8, 128)
    ones16 = jnp.ones((128, 16), F32)
    zeros16 = jnp.zeros((128, 16), F32)
    b1r, b2r, b3r = b1.reshape(1, -1), b2.reshape(1, -1), b3.reshape(1, -1)
    g1r, g2r = g1.reshape(1, -1), g2.reshape(1, -1)
    bt1r, bt2r = bt1.reshape(1, -1), bt2.reshape(1, -1)
    pb1r, pb2r, pb3r = pb1.reshape(1, -1), pb2.reshape(1, -1), pb3.reshape(1, -1)
    pg1r, pg2r = pg1.reshape(1, -1), pg2.reshape(1, -1)
    pbt1r, pbt2r = pbt1.reshape(1, -1), pbt2.reshape(1, -1)

    deg2 = _deg_k(dstr32, ones16, zeros16)
    dis, dis32, u0 = _tc_prep(deg2, xp)

    def to_sc(u, nc):
        return u.reshape(NPAD, nc, 32).transpose(1, 0, 2)

    def to_tc(a):
        return a.transpose(1, 0, 2).reshape(NPAD, -1)

    agg0 = _agg2_k(to_sc(u0, 2), srcr16, dstr16)
    u1 = _fused1(to_tc(agg0), dis, W1, b1r, g1r, bt1r,
                 jnp.zeros((1, 1), F32))

    agg1 = _agg4_k(to_sc(u1, 4), srcr16, dstr16)
    u2 = _fused2(to_tc(agg1), dis, W2, b2r, g2r, bt2r, W3)

    agg2 = _agg2_k(to_sc(u2, 2), srcr16, dstr16)

    gu, gp, du, dp = _gather_k(agg2, dis32, idxr)
    gu64 = gu.transpose(1, 0, 2).reshape(B, 64)
    gp64 = gp.transpose(1, 0, 2).reshape(B, 64)
    out = _tc_pair(gu64, gp64, du, dp, b3r, P1, pb1r, pg1r, pbt1r,
                   P2, pb2r, pg2r, pbt2r, P3, pb3r)
    return out.reshape(B)


# speedup vs baseline: 1.1580x; 1.1580x over previous
import functools

import jax
import jax.numpy as jnp
from jax import lax
from jax.experimental import pallas as pl
from jax.experimental.pallas import tpu as pltpu
from jax.experimental.pallas import tpu_sc as plsc

F32 = jnp.float32

N_NODES = 50000
NPAD = 51200
TROWS = NPAD // 16
N_EDGES = 800000
EPAD = 16 * 392 * 128
DUMMY = N_NODES
B = 16384
USER_OFFSET = 25000

_mesh = plsc.VectorSubcoreMesh(core_axis_name="c", subcore_axis_name="s")
_sc_params = pltpu.CompilerParams(use_tc_tiling_on_sc=False)


def _make_deg():
    @functools.partial(
        pl.kernel,
        out_type=jax.ShapeDtypeStruct((2, NPAD, 16), F32),
        mesh=_mesh,
        compiler_params=_sc_params,
        scratch_types=[
            pltpu.VMEM((196, 128), jnp.int32),
            pltpu.VMEM((128, 16), F32),
            pltpu.VMEM((128, 16), F32),
            pltpu.VMEM_SHARED((NPAD, 16), F32),
        ],
    )
    def deg_k(dstr, ones_hbm, zeros_hbm, out, dst_v, ones_v, zeros_v, acc):
        cid = lax.axis_index("c")
        sid = lax.axis_index("s")
        wid = cid * 16 + sid
        pltpu.sync_copy(dstr.at[wid], dst_v)
        pltpu.sync_copy(ones_hbm, ones_v)
        pltpu.sync_copy(zeros_hbm, zeros_v)
        for z in range(TROWS // 128):
            pltpu.sync_copy(zeros_v, acc.at[pl.ds(sid * TROWS + z * 128, 128)])
        plsc.subcore_barrier()

        def body(j, carry):
            pltpu.sync_copy(ones_v, acc.at[dst_v.at[j]], add=True)
            return carry

        lax.fori_loop(0, 196, body, 0)
        plsc.subcore_barrier()
        pltpu.sync_copy(acc.at[pl.ds(sid * TROWS, TROWS)],
                        out.at[cid].at[pl.ds(sid * TROWS, TROWS)])

    return deg_k


def _make_agg(nc):
    @functools.partial(
        pl.kernel,
        out_type=jax.ShapeDtypeStruct((NPAD, 128), F32),
        mesh=_mesh,
        compiler_params=_sc_params,
        scratch_types=[
            pltpu.VMEM((14, 128), jnp.int32),
            pltpu.VMEM((14, 128), jnp.int32),
            pltpu.VMEM((4, 128, 32), F32),
            pltpu.VMEM_SHARED((NPAD, 32), F32),
            pltpu.SemaphoreType.DMA,
            pltpu.SemaphoreType.DMA,
            pltpu.SemaphoreType.DMA,
            pltpu.SemaphoreType.DMA,
            pltpu.SemaphoreType.DMA,
            pltpu.SemaphoreType.DMA,
            pltpu.SemaphoreType.DMA,
            pltpu.SemaphoreType.DMA,
        ],
    )
    def agg_k(table, srcr, dstr, out,
              src_v, dst_v, rows_v, acc,
              sg0, sg1, sg2, sg3, ss0, ss1, ss2, ss3):
        cid = lax.axis_index("c")
        sid = lax.axis_index("s")
        sg = [sg0, sg1, sg2, sg3]
        ss = [ss0, ss1, ss2, ss3]
        for c in range(nc):
            @pl.when(cid == (c % 2))
            def _():
                pltpu.sync_copy(table.at[c].at[pl.ds(sid * TROWS, TROWS)],
                                acc.at[pl.ds(sid * TROWS, TROWS)])
                plsc.subcore_barrier()

                def body(g, carry):
                    base = g * 14
                    pltpu.sync_copy(srcr.at[sid].at[pl.ds(base, 14)], src_v)
                    pltpu.sync_copy(dstr.at[sid].at[pl.ds(base, 14)], dst_v)
                    gath = [None] * 4
                    for w in range(4):
                        gath[w] = pltpu.async_copy(
                            table.at[c].at[src_v.at[w]], rows_v.at[w], sg[w])
                    tail = []
                    for j in range(14):
                        w = j % 4
                        gath[w].wait()
                        sdesc = pltpu.async_copy(
                            rows_v.at[w], acc.at[dst_v.at[j]], ss[w],
                            add=True)
                        if j + 4 < 14:
                            sdesc.wait()
                            gath[w] = pltpu.async_copy(
                                table.at[c].at[src_v.at[j + 4]],
                                rows_v.at[w], sg[w])
                        else:
                            tail.append(sdesc)
                    for sdesc in tail:
                        sdesc.wait()
                    return carry

                lax.fori_loop(0, 392 // 14, body, 0)
                plsc.subcore_barrier()
                pltpu.sync_copy(
                    acc.at[pl.ds(sid * TROWS, TROWS)],
                    out.at[pl.ds(sid * TROWS, TROWS), pl.ds(32 * c, 32)])

    return agg_k


def _make_gather():
    @functools.partial(
        pl.kernel,
        out_type=[
            jax.ShapeDtypeStruct((2, B, 32), F32),
            jax.ShapeDtypeStruct((2, B, 32), F32),
            jax.ShapeDtypeStruct((B, 32), F32),
            jax.ShapeDtypeStruct((B, 32), F32),
        ],
        mesh=_mesh,
        compiler_params=_sc_params,
        scratch_types=[
            pltpu.VMEM((8, 128), jnp.int32),
            pltpu.VMEM((8, 128), jnp.int32),
            pltpu.VMEM((8, 128), jnp.int32),
            pltpu.VMEM((128, 32), F32),
            pltpu.VMEM((128, 32), F32),
            pltpu.VMEM((128, 32), F32),
            pltpu.SemaphoreType.DMA,
        ],
    )
    def gather_k(agg4v, dis32, idxr, idxra, idxrb, gu, gp, du, dp,
                 idx_v, ia_v, ib_v, r0_v, r1_v, rd_v, sem):
        cid = lax.axis_index("c")
        sid = lax.axis_index("s")
        wid = cid * 16 + sid

        def run(out, outd, base0):
            pltpu.sync_copy(idxr.at[wid], idx_v)
            pltpu.sync_copy(idxra.at[wid], ia_v)
            pltpu.sync_copy(idxrb.at[wid], ib_v)
            for j in range(8):
                base = base0 + j * 128
                pltpu.async_copy(agg4v.at[ia_v.at[j]], r0_v, sem).wait()
                pltpu.sync_copy(r0_v, out.at[0].at[pl.ds(base, 128)])
                pltpu.async_copy(agg4v.at[ib_v.at[j]], r1_v, sem).wait()
                pltpu.sync_copy(r1_v, out.at[1].at[pl.ds(base, 128)])
                pltpu.async_copy(dis32.at[idx_v.at[j]], rd_v, sem).wait()
                pltpu.sync_copy(rd_v, outd.at[pl.ds(base, 128)])

        @pl.when(cid == 0)
        def _():
            run(gu, du, sid * 1024)

        @pl.when(cid == 1)
        def _():
            run(gp, dp, sid * 1024)

    return gather_k


_deg_k = _make_deg()
_agg2_k = _make_agg(2)
_agg4_k = _make_agg(4)
_gather_k = _make_gather()


_BLK = 1600
_NBLK = NPAD // _BLK


def _prep_body(deg2_ref, x_ref, dis_ref, dis32_ref, u0_ref):
    deg = deg2_ref[0, :, 0:1] + deg2_ref[1, :, 0:1] + 1.0
    dis = lax.rsqrt(deg)
    dis_ref[...] = dis
    dis32_ref[...] = jnp.broadcast_to(dis, dis32_ref.shape)
    u0_ref[0] = x_ref[:, 0:32] * dis
    u0_ref[1] = x_ref[:, 32:64] * dis


def _tc_prep(deg2, xp):
    return pl.pallas_call(
        _prep_body,
        grid=(_NBLK,),
        in_specs=[
            pl.BlockSpec((2, _BLK, 16), lambda i: (0, i, 0)),
            pl.BlockSpec((_BLK, 64), lambda i: (i, 0)),
        ],
        out_specs=[
            pl.BlockSpec((_BLK, 1), lambda i: (i, 0)),
            pl.BlockSpec((_BLK, 32), lambda i: (i, 0)),
            pl.BlockSpec((2, _BLK, 32), lambda i: (0, i, 0)),
        ],
        out_shape=[
            jax.ShapeDtypeStruct((NPAD, 1), F32),
            jax.ShapeDtypeStruct((NPAD, 32), F32),
            jax.ShapeDtypeStruct((2, NPAD, 32), F32),
        ],
    )(deg2, xp)


def _make_fused(nc_in, nc_out, with_w3):
    din = 32 * nc_in

    def body(agg_ref, dis_ref, w_ref, b_ref, g_ref, bt_ref, w3_ref,
             u_ref, y_ref, st_ref):
        p = pl.program_id(0)
        i = pl.program_id(1)
        dis = dis_ref[...]

        @pl.when(p == 0)
        def _():
            m = agg_ref[:, 0:din] * dis
            y = jnp.dot(m, w_ref[...], preferred_element_type=F32) \
                + b_ref[...]
            y_ref[pl.ds(i * _BLK, _BLK), :] = y

            @pl.when(i == 0)
            def _():
                st_ref[...] = jnp.zeros_like(st_ref)

            rid = i * _BLK + lax.broadcasted_iota(jnp.int32, (_BLK, 1), 0)
            ym = jnp.where(rid < N_NODES, y, 0.0)
            st_ref[0:1] += jnp.sum(ym, axis=0, keepdims=True)
            st_ref[1:2] += jnp.sum(ym * ym, axis=0, keepdims=True)

        @pl.when(p == 1)
        def _():
            mean = st_ref[0:1] / float(N_NODES)
            var = st_ref[1:2] / float(N_NODES) - mean * mean
            inv = lax.rsqrt(var + 1e-5)
            y = y_ref[pl.ds(i * _BLK, _BLK), :]
            h = jnp.maximum((y - mean) * inv * g_ref[...] + bt_ref[...], 0.0)
            if with_w3:
                h = jnp.dot(h, w3_ref[...], preferred_element_type=F32)
            t = h * dis
            for k in range(nc_out):
                u_ref[k] = t[:, 32 * k:32 * (k + 1)]

    def run(agg, dis, w, b, g, bt, w3):
        dout = w.shape[1]
        return pl.pallas_call(
            body,
            grid=(2, _NBLK),
            in_specs=[
                pl.BlockSpec((_BLK, 128),
                             lambda p, i: (i * (1 - p), 0)),
                pl.BlockSpec((_BLK, 1), lambda p, i: (i, 0)),
                pl.BlockSpec((din, dout), lambda p, i: (0, 0)),
                pl.BlockSpec((1, dout), lambda p, i: (0, 0)),
                pl.BlockSpec((1, dout), lambda p, i: (0, 0)),
                pl.BlockSpec((1, dout), lambda p, i: (0, 0)),
                pl.BlockSpec(w3.shape, lambda p, i: (0, 0)),
            ],
            out_specs=pl.BlockSpec((nc_out, _BLK, 32),
                                   lambda p, i: (0, i, 0)),
            out_shape=jax.ShapeDtypeStruct((nc_out, NPAD, 32), F32),
            scratch_shapes=[
                pltpu.VMEM((NPAD, dout), F32),
                pltpu.VMEM((8, dout), F32),
            ],
            compiler_params=pltpu.CompilerParams(
                vmem_limit_bytes=50 * 1024 * 1024),
        )(agg, dis, w, b, g, bt, w3)

    return run


_fused1 = _make_fused(2, 4, False)
_fused2 = _make_fused(4, 2, True)


def _bn_full(a, g, bt):
    m = jnp.mean(a, axis=0, keepdims=True)
    v = jnp.mean(a * a, axis=0, keepdims=True) - m * m
    return jnp.maximum((a - m) * lax.rsqrt(v + 1e-5) * g + bt, 0.0)


_PBLK = 4096
_PNB = B // _PBLK


def _pair_body(gu_ref, gp_ref, du_ref, dp_ref, b3_ref,
               p1_ref, pb1_ref, pg1_ref, pbt1_ref,
               p2_ref, pb2_ref, pg2_ref, pbt2_ref, p3_ref, pb3_ref,
               out_ref, a1_ref, a2_ref, st1_ref, st2_ref):
    p = pl.program_id(0)
    i = pl.program_id(1)
    rows = pl.ds(i * _PBLK, _PBLK)

    @pl.when(p == 0)
    def _():
        ue = jnp.concatenate([gu_ref[0], gu_ref[1]], axis=1)
        pe = jnp.concatenate([gp_ref[0], gp_ref[1]], axis=1)
        p1t = p1_ref[0:64]
        p1b = p1_ref[64:128]
        bias = (jnp.dot(b3_ref[...], p1t + p1b, preferred_element_type=F32)
                + pb1_ref[...])
        a1 = (du_ref[:, 0:1] * jnp.dot(ue, p1t, preferred_element_type=F32)
              + dp_ref[:, 0:1] * jnp.dot(pe, p1b, preferred_element_type=F32)
              + bias)
        a1_ref[rows, :] = a1

        @pl.when(i == 0)
        def _():
            st1_ref[...] = jnp.zeros_like(st1_ref)

        st1_ref[0:1] += jnp.sum(a1, axis=0, keepdims=True)
        st1_ref[1:2] += jnp.sum(a1 * a1, axis=0, keepdims=True)

    @pl.when(p == 1)
    def _():
        mean = st1_ref[0:1] / float(B)
        var = st1_ref[1:2] / float(B) - mean * mean
        z1 = jnp.maximum((a1_ref[rows, :] - mean) * lax.rsqrt(var + 1e-5)
                         * pg1_ref[...] + pbt1_ref[...], 0.0)
        a2 = jnp.dot(z1, p2_ref[...], preferred_element_type=F32) \
            + pb2_ref[...]
        a2_ref[rows, :] = a2

        @pl.when(i == 0)
        def _():
            st2_ref[...] = jnp.zeros_like(st2_ref)

        st2_ref[0:1] += jnp.sum(a2, axis=0, keepdims=True)
        st2_ref[1:2] += jnp.sum(a2 * a2, axis=0, keepdims=True)

    @pl.when(p == 2)
    def _():
        mean = st2_ref[0:1] / float(B)
        var = st2_ref[1:2] / float(B) - mean * mean
        z2 = jnp.maximum((a2_ref[rows, :] - mean) * lax.rsqrt(var + 1e-5)
                         * pg2_ref[...] + pbt2_ref[...], 0.0)
        a3 = jnp.dot(z2, p3_ref[...], preferred_element_type=F32) \
            + pb3_ref[...]
        out_ref[...] = jax.nn.sigmoid(a3) * 4.0 + 1.0


def _tc_pair(gu, gp, du, dp, b3, p1, pb1, pg1, pbt1,
             p2, pb2, pg2, pbt2, p3, pb3):
    full = lambda shape: pl.BlockSpec(shape, lambda p, i: tuple(
        0 for _ in shape))
    return pl.pallas_call(
        _pair_body,
        grid=(3, _PNB),
        in_specs=[
            pl.BlockSpec((2, _PBLK, 32), lambda p, i: (0, jnp.where(p == 0, i, 0), 0)),
            pl.BlockSpec((2, _PBLK, 32), lambda p, i: (0, jnp.where(p == 0, i, 0), 0)),
            pl.BlockSpec((_PBLK, 32), lambda p, i: (jnp.where(p == 0, i, 0), 0)),
            pl.BlockSpec((_PBLK, 32), lambda p, i: (jnp.where(p == 0, i, 0), 0)),
            full((1, 64)),
            full((128, 128)),
            full((1, 128)),
            full((1, 128)),
            full((1, 128)),
            full((128, 64)),
            full((1, 64)),
            full((1, 64)),
            full((1, 64)),
            full((64, 1)),
            full((1, 1)),
        ],
        out_specs=pl.BlockSpec((_PBLK, 1), lambda p, i: (i, 0)),
        out_shape=jax.ShapeDtypeStruct((B, 1), F32),
        scratch_shapes=[
            pltpu.VMEM((B, 128), F32),
            pltpu.VMEM((B, 64), F32),
            pltpu.VMEM((8, 128), F32),
            pltpu.VMEM((8, 64), F32),
        ],
        compiler_params=pltpu.CompilerParams(
            vmem_limit_bytes=50 * 1024 * 1024),
    )(gu, gp, du, dp, b3, p1, pb1, pg1, pbt1, p2, pb2, pg2, pbt2, p3, pb3)


def kernel(x, edge_index, user_indices, product_indices,
           W1, b1, g1, bt1, W2, b2, g2, bt2, W3, b3,
           P1, pb1, pg1, pbt1, P2, pb2, pg2, pbt2, P3, pb3):
    src = jnp.concatenate(
        [edge_index[0], jnp.zeros((EPAD - N_EDGES,), jnp.int32)])
    dst = jnp.concatenate(
        [edge_index[1], jnp.full((EPAD - N_EDGES,), DUMMY, jnp.int32)])
    srcr16 = src.reshape(16, 392, 128)
    dstr16 = dst.reshape(16, 392, 128)
    dstr32 = dst.reshape(32, 196, 128)
    xp = jnp.pad(x, ((0, NPAD - N_NODES), (0, 0)))
    idx_all = jnp.concatenate([user_indices + USER_OFFSET, product_indices])
    idxr = idx_all.reshape(32, 8, 128)
    idxra = (idx_all * 4).reshape(32, 8, 128)
    idxrb = (idx_all * 4 + 1).reshape(32, 8, 128)
    ones16 = jnp.ones((128, 16), F32)
    zeros16 = jnp.zeros((128, 16), F32)
    b1r, b2r, b3r = b1.reshape(1, -1), b2.reshape(1, -1), b3.reshape(1, -1)
    g1r, g2r = g1.reshape(1, -1), g2.reshape(1, -1)
    bt1r, bt2r = bt1.reshape(1, -1), bt2.reshape(1, -1)
    pb1r, pb2r, pb3r = pb1.reshape(1, -1), pb2.reshape(1, -1), pb3.reshape(1, -1)
    pg1r, pg2r = pg1.reshape(1, -1), pg2.reshape(1, -1)
    pbt1r, pbt2r = pbt1.reshape(1, -1), pbt2.reshape(1, -1)

    deg2 = _deg_k(dstr32, ones16, zeros16)
    dis, dis32, u0 = _tc_prep(deg2, xp)

    agg0 = _agg2_k(u0, srcr16, dstr16)
    u1 = _fused1(agg0, dis, W1, b1r, g1r, bt1r, jnp.zeros((1, 1), F32))

    agg1 = _agg4_k(u1, srcr16, dstr16)
    u2 = _fused2(agg1, dis, W2, b2r, g2r, bt2r, W3)

    agg2 = _agg2_k(u2, srcr16, dstr16)

    gu, gp, du, dp = _gather_k(agg2.reshape(NPAD * 4, 32), dis32,
                               idxr, idxra, idxrb)
    out = _tc_pair(gu, gp, du, dp, b3r, P1, pb1r, pg1r, pbt1r,
                   P2, pb2r, pg2r, pbt2r, P3, pb3r)
    return out.reshape(B)


# speedup vs baseline: 1.2087x; 1.0438x over previous
import functools

import jax
import jax.numpy as jnp
from jax import lax
from jax.experimental import pallas as pl
from jax.experimental.pallas import tpu as pltpu
from jax.experimental.pallas import tpu_sc as plsc

F32 = jnp.float32

N_NODES = 50000
NPAD = 51200
TROWS = NPAD // 16
N_EDGES = 800000
EPAD = 16 * 392 * 128
DUMMY = N_NODES
B = 16384
USER_OFFSET = 25000

_mesh = plsc.VectorSubcoreMesh(core_axis_name="c", subcore_axis_name="s")
_sc_params = pltpu.CompilerParams(use_tc_tiling_on_sc=False)


def _make_deg():
    @functools.partial(
        pl.kernel,
        out_type=jax.ShapeDtypeStruct((2, NPAD, 16), F32),
        mesh=_mesh,
        compiler_params=_sc_params,
        scratch_types=[
            pltpu.VMEM((196, 128), jnp.int32),
            pltpu.VMEM((128, 16), F32),
            pltpu.VMEM((128, 16), F32),
            pltpu.VMEM_SHARED((NPAD, 16), F32),
        ],
    )
    def deg_k(dstr, ones_hbm, zeros_hbm, out, dst_v, ones_v, zeros_v, acc):
        cid = lax.axis_index("c")
        sid = lax.axis_index("s")
        wid = cid * 16 + sid
        pltpu.sync_copy(dstr.at[wid], dst_v)
        pltpu.sync_copy(ones_hbm, ones_v)
        pltpu.sync_copy(zeros_hbm, zeros_v)
        for z in range(TROWS // 128):
            pltpu.sync_copy(zeros_v, acc.at[pl.ds(sid * TROWS + z * 128, 128)])
        plsc.subcore_barrier()

        def body(j, carry):
            pltpu.sync_copy(ones_v, acc.at[dst_v.at[j]], add=True)
            return carry

        lax.fori_loop(0, 196, body, 0)
        plsc.subcore_barrier()
        pltpu.sync_copy(acc.at[pl.ds(sid * TROWS, TROWS)],
                        out.at[cid].at[pl.ds(sid * TROWS, TROWS)])

    return deg_k


def _make_agg(nc):
    @functools.partial(
        pl.kernel,
        out_type=jax.ShapeDtypeStruct((NPAD, 128), F32),
        mesh=_mesh,
        compiler_params=_sc_params,
        scratch_types=[
            pltpu.VMEM((14, 128), jnp.int32),
            pltpu.VMEM((14, 128), jnp.int32),
            pltpu.VMEM((4, 128, 32), F32),
            pltpu.VMEM((TROWS // 128, 128), jnp.int32),
            pltpu.VMEM_SHARED((NPAD, 32), F32),
            pltpu.SemaphoreType.DMA,
            pltpu.SemaphoreType.DMA,
            pltpu.SemaphoreType.DMA,
            pltpu.SemaphoreType.DMA,
            pltpu.SemaphoreType.DMA,
            pltpu.SemaphoreType.DMA,
            pltpu.SemaphoreType.DMA,
            pltpu.SemaphoreType.DMA,
        ],
    )
    def agg_k(tablev, src4r, dstr, init4, out,
              src_v, dst_v, rows_v, init_v, acc,
              sg0, sg1, sg2, sg3, ss0, ss1, ss2, ss3):
        cid = lax.axis_index("c")
        sid = lax.axis_index("s")
        sg = [sg0, sg1, sg2, sg3]
        ss = [ss0, ss1, ss2, ss3]
        for c in range(nc):
            @pl.when(cid == (c % 2))
            def _():
                pltpu.sync_copy(init4.at[c].at[sid], init_v)

                def ibody(z, carry):
                    pltpu.async_copy(
                        tablev.at[init_v.at[z]],
                        rows_v.at[0], sg0).wait()
                    pltpu.sync_copy(
                        rows_v.at[0],
                        acc.at[pl.ds(sid * TROWS + z * 128, 128)])
                    return carry

                lax.fori_loop(0, TROWS // 128, ibody, 0)
                plsc.subcore_barrier()

                def body(g, carry):
                    base = g * 14
                    pltpu.sync_copy(
                        src4r.at[c].at[sid].at[pl.ds(base, 14)], src_v)
                    pltpu.sync_copy(dstr.at[sid].at[pl.ds(base, 14)], dst_v)
                    gath = [None] * 4
                    for w in range(4):
                        gath[w] = pltpu.async_copy(
                            tablev.at[src_v.at[w]], rows_v.at[w], sg[w])
                    tail = []
                    for j in range(14):
                        w = j % 4
                        gath[w].wait()
                        sdesc = pltpu.async_copy(
                            rows_v.at[w], acc.at[dst_v.at[j]], ss[w],
                            add=True)
                        if j + 4 < 14:
                            sdesc.wait()
                            gath[w] = pltpu.async_copy(
                                tablev.at[src_v.at[j + 4]],
                                rows_v.at[w], sg[w])
                        else:
                            tail.append(sdesc)
                    for sdesc in tail:
                        sdesc.wait()
                    return carry

                lax.fori_loop(0, 392 // 14, body, 0)
                plsc.subcore_barrier()
                pltpu.sync_copy(
                    acc.at[pl.ds(sid * TROWS, TROWS)],
                    out.at[pl.ds(sid * TROWS, TROWS), pl.ds(32 * c, 32)])

    return agg_k


def _make_gather():
    @functools.partial(
        pl.kernel,
        out_type=[
            jax.ShapeDtypeStruct((2, B, 32), F32),
            jax.ShapeDtypeStruct((2, B, 32), F32),
            jax.ShapeDtypeStruct((B, 32), F32),
            jax.ShapeDtypeStruct((B, 32), F32),
        ],
        mesh=_mesh,
        compiler_params=_sc_params,
        scratch_types=[
            pltpu.VMEM((8, 128), jnp.int32),
            pltpu.VMEM((8, 128), jnp.int32),
            pltpu.VMEM((8, 128), jnp.int32),
            pltpu.VMEM((128, 32), F32),
            pltpu.VMEM((128, 32), F32),
            pltpu.VMEM((128, 32), F32),
            pltpu.SemaphoreType.DMA,
        ],
    )
    def gather_k(agg4v, dis32, idxr, idxra, idxrb, gu, gp, du, dp,
                 idx_v, ia_v, ib_v, r0_v, r1_v, rd_v, sem):
        cid = lax.axis_index("c")
        sid = lax.axis_index("s")
        wid = cid * 16 + sid

        def run(out, outd, base0):
            pltpu.sync_copy(idxr.at[wid], idx_v)
            pltpu.sync_copy(idxra.at[wid], ia_v)
            pltpu.sync_copy(idxrb.at[wid], ib_v)
            for j in range(8):
                base = base0 + j * 128
                pltpu.async_copy(agg4v.at[ia_v.at[j]], r0_v, sem).wait()
                pltpu.sync_copy(r0_v, out.at[0].at[pl.ds(base, 128)])
                pltpu.async_copy(agg4v.at[ib_v.at[j]], r1_v, sem).wait()
                pltpu.sync_copy(r1_v, out.at[1].at[pl.ds(base, 128)])
                pltpu.async_copy(dis32.at[idx_v.at[j]], rd_v, sem).wait()
                pltpu.sync_copy(rd_v, outd.at[pl.ds(base, 128)])

        @pl.when(cid == 0)
        def _():
            run(gu, du, sid * 1024)

        @pl.when(cid == 1)
        def _():
            run(gp, dp, sid * 1024)

    return gather_k


_deg_k = _make_deg()
_agg2_k = _make_agg(2)
_agg4_k = _make_agg(4)
_gather_k = _make_gather()


_BLK = 1600
_NBLK = NPAD // _BLK


def _prep_body(deg2_ref, x_ref, dis_ref, dis32_ref, u0_ref):
    deg = deg2_ref[0, :, 0:1] + deg2_ref[1, :, 0:1] + 1.0
    dis = lax.rsqrt(deg)
    dis_ref[...] = dis
    dis32_ref[...] = jnp.broadcast_to(dis, dis32_ref.shape)
    u0_ref[...] = jnp.concatenate(
        [x_ref[...] * dis, jnp.zeros_like(x_ref)], axis=1)


def _tc_prep(deg2, xp):
    return pl.pallas_call(
        _prep_body,
        grid=(_NBLK,),
        in_specs=[
            pl.BlockSpec((2, _BLK, 16), lambda i: (0, i, 0)),
            pl.BlockSpec((_BLK, 64), lambda i: (i, 0)),
        ],
        out_specs=[
            pl.BlockSpec((_BLK, 1), lambda i: (i, 0)),
            pl.BlockSpec((_BLK, 32), lambda i: (i, 0)),
            pl.BlockSpec((_BLK, 128), lambda i: (i, 0)),
        ],
        out_shape=[
            jax.ShapeDtypeStruct((NPAD, 1), F32),
            jax.ShapeDtypeStruct((NPAD, 32), F32),
            jax.ShapeDtypeStruct((NPAD, 128), F32),
        ],
    )(deg2, xp)


def _make_fused(nc_in, nc_out, with_w3):
    din = 32 * nc_in

    def body(agg_ref, dis_ref, w_ref, b_ref, g_ref, bt_ref, w3_ref,
             u_ref, y_ref, st_ref):
        p = pl.program_id(0)
        i = pl.program_id(1)
        dis = dis_ref[...]

        @pl.when(p == 0)
        def _():
            m = agg_ref[:, 0:din] * dis
            y = jnp.dot(m, w_ref[...], preferred_element_type=F32) \
                + b_ref[...]
            y_ref[pl.ds(i * _BLK, _BLK), :] = y

            @pl.when(i == 0)
            def _():
                st_ref[...] = jnp.zeros_like(st_ref)

            rid = i * _BLK + lax.broadcasted_iota(jnp.int32, (_BLK, 1), 0)
            ym = jnp.where(rid < N_NODES, y, 0.0)
            st_ref[0:1] += jnp.sum(ym, axis=0, keepdims=True)
            st_ref[1:2] += jnp.sum(ym * ym, axis=0, keepdims=True)

        @pl.when(p == 1)
        def _():
            mean = st_ref[0:1] / float(N_NODES)
            var = st_ref[1:2] / float(N_NODES) - mean * mean
            inv = lax.rsqrt(var + 1e-5)
            y = y_ref[pl.ds(i * _BLK, _BLK), :]
            h = jnp.maximum((y - mean) * inv * g_ref[...] + bt_ref[...], 0.0)
            if with_w3:
                h = jnp.dot(h, w3_ref[...], preferred_element_type=F32)
            t = h * dis
            if 32 * nc_out < 128:
                t = jnp.concatenate(
                    [t, jnp.zeros((t.shape[0], 128 - 32 * nc_out), F32)],
                    axis=1)
            u_ref[...] = t

    def run(agg, dis, w, b, g, bt, w3):
        dout = w.shape[1]
        return pl.pallas_call(
            body,
            grid=(2, _NBLK),
            in_specs=[
                pl.BlockSpec((_BLK, 128),
                             lambda p, i: (i * (1 - p), 0)),
                pl.BlockSpec((_BLK, 1), lambda p, i: (i, 0)),
                pl.BlockSpec((din, dout), lambda p, i: (0, 0)),
                pl.BlockSpec((1, dout), lambda p, i: (0, 0)),
                pl.BlockSpec((1, dout), lambda p, i: (0, 0)),
                pl.BlockSpec((1, dout), lambda p, i: (0, 0)),
                pl.BlockSpec(w3.shape, lambda p, i: (0, 0)),
            ],
            out_specs=pl.BlockSpec((_BLK, 128), lambda p, i: (i, 0)),
            out_shape=jax.ShapeDtypeStruct((NPAD, 128), F32),
            scratch_shapes=[
                pltpu.VMEM((NPAD, dout), F32),
                pltpu.VMEM((8, dout), F32),
            ],
            compiler_params=pltpu.CompilerParams(
                vmem_limit_bytes=50 * 1024 * 1024),
        )(agg, dis, w, b, g, bt, w3)

    return run


_fused1 = _make_fused(2, 4, False)
_fused2 = _make_fused(4, 2, True)


def _bn_full(a, g, bt):
    m = jnp.mean(a, axis=0, keepdims=True)
    v = jnp.mean(a * a, axis=0, keepdims=True) - m * m
    return jnp.maximum((a - m) * lax.rsqrt(v + 1e-5) * g + bt, 0.0)


_PBLK = 4096
_PNB = B // _PBLK


def _pair_body(gu_ref, gp_ref, du_ref, dp_ref, b3_ref,
               p1_ref, pb1_ref, pg1_ref, pbt1_ref,
               p2_ref, pb2_ref, pg2_ref, pbt2_ref, p3_ref, pb3_ref,
               out_ref, a1_ref, a2_ref, st1_ref, st2_ref):
    p = pl.program_id(0)
    i = pl.program_id(1)
    rows = pl.ds(i * _PBLK, _PBLK)

    @pl.when(p == 0)
    def _():
        ue = jnp.concatenate([gu_ref[0], gu_ref[1]], axis=1)
        pe = jnp.concatenate([gp_ref[0], gp_ref[1]], axis=1)
        p1t = p1_ref[0:64]
        p1b = p1_ref[64:128]
        bias = (jnp.dot(b3_ref[...], p1t + p1b, preferred_element_type=F32)
                + pb1_ref[...])
        a1 = (du_ref[:, 0:1] * jnp.dot(ue, p1t, preferred_element_type=F32)
              + dp_ref[:, 0:1] * jnp.dot(pe, p1b, preferred_element_type=F32)
              + bias)
        a1_ref[rows, :] = a1

        @pl.when(i == 0)
        def _():
            st1_ref[...] = jnp.zeros_like(st1_ref)

        st1_ref[0:1] += jnp.sum(a1, axis=0, keepdims=True)
        st1_ref[1:2] += jnp.sum(a1 * a1, axis=0, keepdims=True)

    @pl.when(p == 1)
    def _():
        mean = st1_ref[0:1] / float(B)
        var = st1_ref[1:2] / float(B) - mean * mean
        z1 = jnp.maximum((a1_ref[rows, :] - mean) * lax.rsqrt(var + 1e-5)
                         * pg1_ref[...] + pbt1_ref[...], 0.0)
        a2 = jnp.dot(z1, p2_ref[...], preferred_element_type=F32) \
            + pb2_ref[...]
        a2_ref[rows, :] = a2

        @pl.when(i == 0)
        def _():
            st2_ref[...] = jnp.zeros_like(st2_ref)

        st2_ref[0:1] += jnp.sum(a2, axis=0, keepdims=True)
        st2_ref[1:2] += jnp.sum(a2 * a2, axis=0, keepdims=True)

    @pl.when(p == 2)
    def _():
        mean = st2_ref[0:1] / float(B)
        var = st2_ref[1:2] / float(B) - mean * mean
        z2 = jnp.maximum((a2_ref[rows, :] - mean) * lax.rsqrt(var + 1e-5)
                         * pg2_ref[...] + pbt2_ref[...], 0.0)
        a3 = jnp.dot(z2, p3_ref[...], preferred_element_type=F32) \
            + pb3_ref[...]
        out_ref[...] = jax.nn.sigmoid(a3) * 4.0 + 1.0


def _tc_pair(gu, gp, du, dp, b3, p1, pb1, pg1, pbt1,
             p2, pb2, pg2, pbt2, p3, pb3):
    full = lambda shape: pl.BlockSpec(shape, lambda p, i: tuple(
        0 for _ in shape))
    return pl.pallas_call(
        _pair_body,
        grid=(3, _PNB),
        in_specs=[
            pl.BlockSpec((2, _PBLK, 32), lambda p, i: (0, jnp.where(p == 0, i, 0), 0)),
            pl.BlockSpec((2, _PBLK, 32), lambda p, i: (0, jnp.where(p == 0, i, 0), 0)),
            pl.BlockSpec((_PBLK, 32), lambda p, i: (jnp.where(p == 0, i, 0), 0)),
            pl.BlockSpec((_PBLK, 32), lambda p, i: (jnp.where(p == 0, i, 0), 0)),
            full((1, 64)),
            full((128, 128)),
            full((1, 128)),
            full((1, 128)),
            full((1, 128)),
            full((128, 64)),
            full((1, 64)),
            full((1, 64)),
            full((1, 64)),
            full((64, 1)),
            full((1, 1)),
        ],
        out_specs=pl.BlockSpec((_PBLK, 1), lambda p, i: (i, 0)),
        out_shape=jax.ShapeDtypeStruct((B, 1), F32),
        scratch_shapes=[
            pltpu.VMEM((B, 128), F32),
            pltpu.VMEM((B, 64), F32),
            pltpu.VMEM((8, 128), F32),
            pltpu.VMEM((8, 64), F32),
        ],
        compiler_params=pltpu.CompilerParams(
            vmem_limit_bytes=50 * 1024 * 1024),
    )(gu, gp, du, dp, b3, p1, pb1, pg1, pbt1, p2, pb2, pg2, pbt2, p3, pb3)


def kernel(x, edge_index, user_indices, product_indices,
           W1, b1, g1, bt1, W2, b2, g2, bt2, W3, b3,
           P1, pb1, pg1, pbt1, P2, pb2, pg2, pbt2, P3, pb3):
    src = jnp.concatenate(
        [edge_index[0], jnp.zeros((EPAD - N_EDGES,), jnp.int32)])
    dst = jnp.concatenate(
        [edge_index[1], jnp.full((EPAD - N_EDGES,), DUMMY, jnp.int32)])
    src4r = ((src * 4)[None, :]
             + jnp.arange(4, dtype=jnp.int32)[:, None]).reshape(
                 4, 16, 392, 128)
    init4 = ((jnp.arange(NPAD, dtype=jnp.int32) * 4)[None, :]
             + jnp.arange(4, dtype=jnp.int32)[:, None]).reshape(
                 4, 16, TROWS // 128, 128)
    dstr16 = dst.reshape(16, 392, 128)
    dstr32 = dst.reshape(32, 196, 128)
    xp = jnp.pad(x, ((0, NPAD - N_NODES), (0, 0)))
    idx_all = jnp.concatenate([user_indices + USER_OFFSET, product_indices])
    idxr = idx_all.reshape(32, 8, 128)
    idxra = (idx_all * 4).reshape(32, 8, 128)
    idxrb = (idx_all * 4 + 1).reshape(32, 8, 128)
    ones16 = jnp.ones((128, 16), F32)
    zeros16 = jnp.zeros((128, 16), F32)
    b1r, b2r, b3r = b1.reshape(1, -1), b2.reshape(1, -1), b3.reshape(1, -1)
    g1r, g2r = g1.reshape(1, -1), g2.reshape(1, -1)
    bt1r, bt2r = bt1.reshape(1, -1), bt2.reshape(1, -1)
    pb1r, pb2r, pb3r = pb1.reshape(1, -1), pb2.reshape(1, -1), pb3.reshape(1, -1)
    pg1r, pg2r = pg1.reshape(1, -1), pg2.reshape(1, -1)
    pbt1r, pbt2r = pbt1.reshape(1, -1), pbt2.reshape(1, -1)

    deg2 = _deg_k(dstr32, ones16, zeros16)
    dis, dis32, u0 = _tc_prep(deg2, xp)

    agg0 = _agg2_k(u0.reshape(NPAD * 4, 32), src4r, dstr16, init4)
    u1 = _fused1(agg0, dis, W1, b1r, g1r, bt1r, jnp.zeros((1, 1), F32))

    agg1 = _agg4_k(u1.reshape(NPAD * 4, 32), src4r, dstr16, init4)
    u2 = _fused2(agg1, dis, W2, b2r, g2r, bt2r, W3)

    agg2 = _agg2_k(u2.reshape(NPAD * 4, 32), src4r, dstr16, init4)

    gu, gp, du, dp = _gather_k(agg2.reshape(NPAD * 4, 32), dis32,
                               idxr, idxra, idxrb)
    out = _tc_pair(gu, gp, du, dp, b3r, P1, pb1r, pg1r, pbt1r,
                   P2, pb2r, pg2r, pbt2r, P3, pb3r)
    return out.reshape(B)


# speedup vs baseline: 1.2908x; 1.0679x over previous
import functools

import jax
import jax.numpy as jnp
from jax import lax
from jax.experimental import pallas as pl
from jax.experimental.pallas import tpu as pltpu
from jax.experimental.pallas import tpu_sc as plsc

F32 = jnp.float32

N_NODES = 50000
NPAD = 51200
TROWS = NPAD // 16
N_EDGES = 800000
EPAD = 16 * 392 * 128
DUMMY = N_NODES
B = 16384
USER_OFFSET = 25000

_mesh = plsc.VectorSubcoreMesh(core_axis_name="c", subcore_axis_name="s")
_sc_params = pltpu.CompilerParams(use_tc_tiling_on_sc=False)


def _make_deg():
    @functools.partial(
        pl.kernel,
        out_type=jax.ShapeDtypeStruct((2, NPAD, 16), F32),
        mesh=_mesh,
        compiler_params=_sc_params,
        scratch_types=[
            pltpu.VMEM((196, 128), jnp.int32),
            pltpu.VMEM((128, 16), F32),
            pltpu.VMEM((128, 16), F32),
            pltpu.VMEM_SHARED((NPAD, 16), F32),
        ],
    )
    def deg_k(dstr, ones_hbm, zeros_hbm, out, dst_v, ones_v, zeros_v, acc):
        cid = lax.axis_index("c")
        sid = lax.axis_index("s")
        wid = cid * 16 + sid
        pltpu.sync_copy(dstr.at[wid], dst_v)
        pltpu.sync_copy(ones_hbm, ones_v)
        pltpu.sync_copy(zeros_hbm, zeros_v)
        for z in range(TROWS // 128):
            pltpu.sync_copy(zeros_v, acc.at[pl.ds(sid * TROWS + z * 128, 128)])
        plsc.subcore_barrier()

        def body(j, carry):
            pltpu.sync_copy(ones_v, acc.at[dst_v.at[j]], add=True)
            return carry

        lax.fori_loop(0, 196, body, 0)
        plsc.subcore_barrier()
        pltpu.sync_copy(acc.at[pl.ds(sid * TROWS, TROWS)],
                        out.at[cid].at[pl.ds(sid * TROWS, TROWS)])

    return deg_k


def _make_agg(nc):
    @functools.partial(
        pl.kernel,
        out_type=jax.ShapeDtypeStruct((NPAD, 128), F32),
        mesh=_mesh,
        compiler_params=_sc_params,
        scratch_types=[
            pltpu.VMEM((28, 128), jnp.int32),
            pltpu.VMEM((28, 128), jnp.int32),
            pltpu.VMEM((4, 128, 32), F32),
            pltpu.VMEM((TROWS // 128, 128), jnp.int32),
            pltpu.VMEM_SHARED((NPAD, 32), F32),
            pltpu.SemaphoreType.DMA,
            pltpu.SemaphoreType.DMA,
            pltpu.SemaphoreType.DMA,
            pltpu.SemaphoreType.DMA,
            pltpu.SemaphoreType.DMA,
            pltpu.SemaphoreType.DMA,
            pltpu.SemaphoreType.DMA,
            pltpu.SemaphoreType.DMA,
        ],
    )
    def agg_k(tablev, src4r, dstr, init4, out,
              src_v, dst_v, rows_v, init_v, acc,
              sg0, sg1, sg2, sg3, ss0, ss1, ss2, ss3):
        cid = lax.axis_index("c")
        sid = lax.axis_index("s")
        sg = [sg0, sg1, sg2, sg3]
        ss = [ss0, ss1, ss2, ss3]
        for c in range(nc):
            @pl.when(cid == (c % 2))
            def _():
                pltpu.sync_copy(init4.at[c].at[sid], init_v)

                def ibody(z, carry):
                    pltpu.async_copy(
                        tablev.at[init_v.at[z]],
                        rows_v.at[0], sg0).wait()
                    pltpu.sync_copy(
                        rows_v.at[0],
                        acc.at[pl.ds(sid * TROWS + z * 128, 128)])
                    return carry

                lax.fori_loop(0, TROWS // 128, ibody, 0)
                plsc.subcore_barrier()

                def body(g, carry):
                    base = g * 28
                    pltpu.sync_copy(
                        src4r.at[c].at[sid].at[pl.ds(base, 28)], src_v)
                    pltpu.sync_copy(dstr.at[sid].at[pl.ds(base, 28)], dst_v)
                    gath = [None] * 4
                    for w in range(4):
                        gath[w] = pltpu.async_copy(
                            tablev.at[src_v.at[w]], rows_v.at[w], sg[w])
                    tail = []
                    for j in range(28):
                        w = j % 4
                        gath[w].wait()
                        sdesc = pltpu.async_copy(
                            rows_v.at[w], acc.at[dst_v.at[j]], ss[w],
                            add=True)
                        if j + 4 < 28:
                            sdesc.wait()
                            gath[w] = pltpu.async_copy(
                                tablev.at[src_v.at[j + 4]],
                                rows_v.at[w], sg[w])
                        else:
                            tail.append(sdesc)
                    for sdesc in tail:
                        sdesc.wait()
                    return carry

                lax.fori_loop(0, 392 // 28, body, 0)
                plsc.subcore_barrier()
                pltpu.sync_copy(
                    acc.at[pl.ds(sid * TROWS, TROWS)],
                    out.at[pl.ds(sid * TROWS, TROWS), pl.ds(32 * c, 32)])

    return agg_k


def _make_gather():
    @functools.partial(
        pl.kernel,
        out_type=[
            jax.ShapeDtypeStruct((2, B, 32), F32),
            jax.ShapeDtypeStruct((2, B, 32), F32),
            jax.ShapeDtypeStruct((B, 32), F32),
            jax.ShapeDtypeStruct((B, 32), F32),
        ],
        mesh=_mesh,
        compiler_params=_sc_params,
        scratch_types=[
            pltpu.VMEM((8, 128), jnp.int32),
            pltpu.VMEM((8, 128), jnp.int32),
            pltpu.VMEM((8, 128), jnp.int32),
            pltpu.VMEM((128, 32), F32),
            pltpu.VMEM((128, 32), F32),
            pltpu.VMEM((128, 32), F32),
            pltpu.SemaphoreType.DMA,
        ],
    )
    def gather_k(agg4v, dis32, idxr, idxra, idxrb, gu, gp, du, dp,
                 idx_v, ia_v, ib_v, r0_v, r1_v, rd_v, sem):
        cid = lax.axis_index("c")
        sid = lax.axis_index("s")
        wid = cid * 16 + sid

        def run(out, outd, base0):
            pltpu.sync_copy(idxr.at[wid], idx_v)
            pltpu.sync_copy(idxra.at[wid], ia_v)
            pltpu.sync_copy(idxrb.at[wid], ib_v)
            for j in range(8):
                base = base0 + j * 128
                pltpu.async_copy(agg4v.at[ia_v.at[j]], r0_v, sem).wait()
                pltpu.sync_copy(r0_v, out.at[0].at[pl.ds(base, 128)])
                pltpu.async_copy(agg4v.at[ib_v.at[j]], r1_v, sem).wait()
                pltpu.sync_copy(r1_v, out.at[1].at[pl.ds(base, 128)])
                pltpu.async_copy(dis32.at[idx_v.at[j]], rd_v, sem).wait()
                pltpu.sync_copy(rd_v, outd.at[pl.ds(base, 128)])

        @pl.when(cid == 0)
        def _():
            run(gu, du, sid * 1024)

        @pl.when(cid == 1)
        def _():
            run(gp, dp, sid * 1024)

    return gather_k


_deg_k = _make_deg()
_agg2_k = _make_agg(2)
_agg4_k = _make_agg(4)
_gather_k = _make_gather()


_BLK = 1600
_NBLK = NPAD // _BLK


def _prep_body(deg2_ref, x_ref, dis_ref, dis32_ref, u0_ref):
    deg = deg2_ref[0, :, 0:1] + deg2_ref[1, :, 0:1] + 1.0
    dis = lax.rsqrt(deg)
    dis_ref[...] = dis
    dis32_ref[...] = jnp.broadcast_to(dis, dis32_ref.shape)
    u0_ref[...] = jnp.concatenate(
        [x_ref[...] * dis, jnp.zeros_like(x_ref)], axis=1)


def _tc_prep(deg2, xp):
    return pl.pallas_call(
        _prep_body,
        grid=(_NBLK,),
        in_specs=[
            pl.BlockSpec((2, _BLK, 16), lambda i: (0, i, 0)),
            pl.BlockSpec((_BLK, 64), lambda i: (i, 0)),
        ],
        out_specs=[
            pl.BlockSpec((_BLK, 1), lambda i: (i, 0)),
            pl.BlockSpec((_BLK, 32), lambda i: (i, 0)),
            pl.BlockSpec((_BLK, 128), lambda i: (i, 0)),
        ],
        out_shape=[
            jax.ShapeDtypeStruct((NPAD, 1), F32),
            jax.ShapeDtypeStruct((NPAD, 32), F32),
            jax.ShapeDtypeStruct((NPAD, 128), F32),
        ],
    )(deg2, xp)


def _make_fused(nc_in, nc_out, with_w3):
    din = 32 * nc_in

    def body(agg_ref, dis_ref, w_ref, b_ref, g_ref, bt_ref, w3_ref,
             u_ref, y_ref, st_ref):
        p = pl.program_id(0)
        i = pl.program_id(1)
        dis = dis_ref[...]

        @pl.when(p == 0)
        def _():
            m = agg_ref[:, 0:din] * dis
            y = jnp.dot(m, w_ref[...], preferred_element_type=F32) \
                + b_ref[...]
            y_ref[pl.ds(i * _BLK, _BLK), :] = y

            @pl.when(i == 0)
            def _():
                st_ref[...] = jnp.zeros_like(st_ref)

            rid = i * _BLK + lax.broadcasted_iota(jnp.int32, (_BLK, 1), 0)
            ym = jnp.where(rid < N_NODES, y, 0.0)
            st_ref[0:1] += jnp.sum(ym, axis=0, keepdims=True)
            st_ref[1:2] += jnp.sum(ym * ym, axis=0, keepdims=True)

        @pl.when(p == 1)
        def _():
            mean = st_ref[0:1] / float(N_NODES)
            var = st_ref[1:2] / float(N_NODES) - mean * mean
            inv = lax.rsqrt(var + 1e-5)
            y = y_ref[pl.ds(i * _BLK, _BLK), :]
            h = jnp.maximum((y - mean) * inv * g_ref[...] + bt_ref[...], 0.0)
            if with_w3:
                h = jnp.dot(h, w3_ref[...], preferred_element_type=F32)
            t = h * dis
            if 32 * nc_out < 128:
                t = jnp.concatenate(
                    [t, jnp.zeros((t.shape[0], 128 - 32 * nc_out), F32)],
                    axis=1)
            u_ref[...] = t

    def run(agg, dis, w, b, g, bt, w3):
        dout = w.shape[1]
        return pl.pallas_call(
            body,
            grid=(2, _NBLK),
            in_specs=[
                pl.BlockSpec((_BLK, 128),
                             lambda p, i: (i * (1 - p), 0)),
                pl.BlockSpec((_BLK, 1), lambda p, i: (i, 0)),
                pl.BlockSpec((din, dout), lambda p, i: (0, 0)),
                pl.BlockSpec((1, dout), lambda p, i: (0, 0)),
                pl.BlockSpec((1, dout), lambda p, i: (0, 0)),
                pl.BlockSpec((1, dout), lambda p, i: (0, 0)),
                pl.BlockSpec(w3.shape, lambda p, i: (0, 0)),
            ],
            out_specs=pl.BlockSpec((_BLK, 128), lambda p, i: (i, 0)),
            out_shape=jax.ShapeDtypeStruct((NPAD, 128), F32),
            scratch_shapes=[
                pltpu.VMEM((NPAD, dout), F32),
                pltpu.VMEM((8, dout), F32),
            ],
            compiler_params=pltpu.CompilerParams(
                vmem_limit_bytes=50 * 1024 * 1024),
        )(agg, dis, w, b, g, bt, w3)

    return run


_fused1 = _make_fused(2, 4, False)
_fused2 = _make_fused(4, 2, True)


def _bn_full(a, g, bt):
    m = jnp.mean(a, axis=0, keepdims=True)
    v = jnp.mean(a * a, axis=0, keepdims=True) - m * m
    return jnp.maximum((a - m) * lax.rsqrt(v + 1e-5) * g + bt, 0.0)


_PBLK = 4096
_PNB = B // _PBLK


def _pair_body(gu_ref, gp_ref, du_ref, dp_ref, b3_ref,
               p1_ref, pb1_ref, pg1_ref, pbt1_ref,
               p2_ref, pb2_ref, pg2_ref, pbt2_ref, p3_ref, pb3_ref,
               out_ref, a1_ref, a2_ref, st1_ref, st2_ref):
    p = pl.program_id(0)
    i = pl.program_id(1)
    rows = pl.ds(i * _PBLK, _PBLK)

    @pl.when(p == 0)
    def _():
        ue = jnp.concatenate([gu_ref[0], gu_ref[1]], axis=1)
        pe = jnp.concatenate([gp_ref[0], gp_ref[1]], axis=1)
        p1t = p1_ref[0:64]
        p1b = p1_ref[64:128]
        bias = (jnp.dot(b3_ref[...], p1t + p1b, preferred_element_type=F32)
                + pb1_ref[...])
        a1 = (du_ref[:, 0:1] * jnp.dot(ue, p1t, preferred_element_type=F32)
              + dp_ref[:, 0:1] * jnp.dot(pe, p1b, preferred_element_type=F32)
              + bias)
        a1_ref[rows, :] = a1

        @pl.when(i == 0)
        def _():
            st1_ref[...] = jnp.zeros_like(st1_ref)

        st1_ref[0:1] += jnp.sum(a1, axis=0, keepdims=True)
        st1_ref[1:2] += jnp.sum(a1 * a1, axis=0, keepdims=True)

    @pl.when(p == 1)
    def _():
        mean = st1_ref[0:1] / float(B)
        var = st1_ref[1:2] / float(B) - mean * mean
        z1 = jnp.maximum((a1_ref[rows, :] - mean) * lax.rsqrt(var + 1e-5)
                         * pg1_ref[...] + pbt1_ref[...], 0.0)
        a2 = jnp.dot(z1, p2_ref[...], preferred_element_type=F32) \
            + pb2_ref[...]
        a2_ref[rows, :] = a2

        @pl.when(i == 0)
        def _():
            st2_ref[...] = jnp.zeros_like(st2_ref)

        st2_ref[0:1] += jnp.sum(a2, axis=0, keepdims=True)
        st2_ref[1:2] += jnp.sum(a2 * a2, axis=0, keepdims=True)

    @pl.when(p == 2)
    def _():
        mean = st2_ref[0:1] / float(B)
        var = st2_ref[1:2] / float(B) - mean * mean
        z2 = jnp.maximum((a2_ref[rows, :] - mean) * lax.rsqrt(var + 1e-5)
                         * pg2_ref[...] + pbt2_ref[...], 0.0)
        a3 = jnp.dot(z2, p3_ref[...], preferred_element_type=F32) \
            + pb3_ref[...]
        out_ref[...] = jax.nn.sigmoid(a3) * 4.0 + 1.0


def _tc_pair(gu, gp, du, dp, b3, p1, pb1, pg1, pbt1,
             p2, pb2, pg2, pbt2, p3, pb3):
    full = lambda shape: pl.BlockSpec(shape, lambda p, i: tuple(
        0 for _ in shape))
    return pl.pallas_call(
        _pair_body,
        grid=(3, _PNB),
        in_specs=[
            pl.BlockSpec((2, _PBLK, 32), lambda p, i: (0, jnp.where(p == 0, i, 0), 0)),
            pl.BlockSpec((2, _PBLK, 32), lambda p, i: (0, jnp.where(p == 0, i, 0), 0)),
            pl.BlockSpec((_PBLK, 32), lambda p, i: (jnp.where(p == 0, i, 0), 0)),
            pl.BlockSpec((_PBLK, 32), lambda p, i: (jnp.where(p == 0, i, 0), 0)),
            full((1, 64)),
            full((128, 128)),
            full((1, 128)),
            full((1, 128)),
            full((1, 128)),
            full((128, 64)),
            full((1, 64)),
            full((1, 64)),
            full((1, 64)),
            full((64, 1)),
            full((1, 1)),
        ],
        out_specs=pl.BlockSpec((_PBLK, 1), lambda p, i: (i, 0)),
        out_shape=jax.ShapeDtypeStruct((B, 1), F32),
        scratch_shapes=[
            pltpu.VMEM((B, 128), F32),
            pltpu.VMEM((B, 64), F32),
            pltpu.VMEM((8, 128), F32),
            pltpu.VMEM((8, 64), F32),
        ],
        compiler_params=pltpu.CompilerParams(
            vmem_limit_bytes=50 * 1024 * 1024),
    )(gu, gp, du, dp, b3, p1, pb1, pg1, pbt1, p2, pb2, pg2, pbt2, p3, pb3)


def kernel(x, edge_index, user_indices, product_indices,
           W1, b1, g1, bt1, W2, b2, g2, bt2, W3, b3,
           P1, pb1, pg1, pbt1, P2, pb2, pg2, pbt2, P3, pb3):
    src = jnp.concatenate(
        [edge_index[0], jnp.zeros((EPAD - N_EDGES,), jnp.int32)])
    dst = jnp.concatenate(
        [edge_index[1], jnp.full((EPAD - N_EDGES,), DUMMY, jnp.int32)])
    src4r = ((src * 4)[None, :]
             + jnp.arange(4, dtype=jnp.int32)[:, None]).reshape(
                 4, 16, 392, 128)
    init4 = ((jnp.arange(NPAD, dtype=jnp.int32) * 4)[None, :]
             + jnp.arange(4, dtype=jnp.int32)[:, None]).reshape(
                 4, 16, TROWS // 128, 128)
    dstr16 = dst.reshape(16, 392, 128)
    dstr32 = dst.reshape(32, 196, 128)
    xp = jnp.pad(x, ((0, NPAD - N_NODES), (0, 0)))
    idx_all = jnp.concatenate([user_indices + USER_OFFSET, product_indices])
    idxr = idx_all.reshape(32, 8, 128)
    idxra = (idx_all * 4).reshape(32, 8, 128)
    idxrb = (idx_all * 4 + 1).reshape(32, 8, 128)
    ones16 = jnp.ones((128, 16), F32)
    zeros16 = jnp.zeros((128, 16), F32)
    b1r, b2r, b3r = b1.reshape(1, -1), b2.reshape(1, -1), b3.reshape(1, -1)
    g1r, g2r = g1.reshape(1, -1), g2.reshape(1, -1)
    bt1r, bt2r = bt1.reshape(1, -1), bt2.reshape(1, -1)
    pb1r, pb2r, pb3r = pb1.reshape(1, -1), pb2.reshape(1, -1), pb3.reshape(1, -1)
    pg1r, pg2r = pg1.reshape(1, -1), pg2.reshape(1, -1)
    pbt1r, pbt2r = pbt1.reshape(1, -1), pbt2.reshape(1, -1)

    deg2 = _deg_k(dstr32, ones16, zeros16)
    dis, dis32, u0 = _tc_prep(deg2, xp)

    agg0 = _agg2_k(u0.reshape(NPAD * 4, 32), src4r, dstr16, init4)
    u1 = _fused1(agg0, dis, W1, b1r, g1r, bt1r, jnp.zeros((1, 1), F32))

    agg1 = _agg4_k(u1.reshape(NPAD * 4, 32), src4r, dstr16, init4)
    u2 = _fused2(agg1, dis, W2, b2r, g2r, bt2r, W3)

    agg2 = _agg2_k(u2.reshape(NPAD * 4, 32), src4r, dstr16, init4)

    gu, gp, du, dp = _gather_k(agg2.reshape(NPAD * 4, 32), dis32,
                               idxr, idxra, idxrb)
    out = _tc_pair(gu, gp, du, dp, b3r, P1, pb1r, pg1r, pbt1r,
                   P2, pb2r, pg2r, pbt2r, P3, pb3r)
    return out.reshape(B)


# speedup vs baseline: 1.3013x; 1.0081x over previous
import functools

import jax
import jax.numpy as jnp
from jax import lax
from jax.experimental import pallas as pl
from jax.experimental.pallas import tpu as pltpu
from jax.experimental.pallas import tpu_sc as plsc

F32 = jnp.float32

N_NODES = 50000
NPAD = 51200
TROWS = NPAD // 16
N_EDGES = 800000
EPAD = 16 * 392 * 128
DUMMY = N_NODES
B = 16384
USER_OFFSET = 25000

_mesh = plsc.VectorSubcoreMesh(core_axis_name="c", subcore_axis_name="s")
_sc_params = pltpu.CompilerParams(use_tc_tiling_on_sc=False)


def _make_deg():
    @functools.partial(
        pl.kernel,
        out_type=jax.ShapeDtypeStruct((2, NPAD, 16), F32),
        mesh=_mesh,
        compiler_params=_sc_params,
        scratch_types=[
            pltpu.VMEM((196, 128), jnp.int32),
            pltpu.VMEM((128, 16), F32),
            pltpu.VMEM((128, 16), F32),
            pltpu.VMEM_SHARED((NPAD, 16), F32),
        ],
    )
    def deg_k(dstr, ones_hbm, zeros_hbm, out, dst_v, ones_v, zeros_v, acc):
        cid = lax.axis_index("c")
        sid = lax.axis_index("s")
        wid = cid * 16 + sid
        pltpu.sync_copy(dstr.at[wid], dst_v)
        pltpu.sync_copy(ones_hbm, ones_v)
        pltpu.sync_copy(zeros_hbm, zeros_v)
        for z in range(TROWS // 128):
            pltpu.sync_copy(zeros_v, acc.at[pl.ds(sid * TROWS + z * 128, 128)])
        plsc.subcore_barrier()

        def body(j, carry):
            pltpu.sync_copy(ones_v, acc.at[dst_v.at[j]], add=True)
            return carry

        lax.fori_loop(0, 196, body, 0)
        plsc.subcore_barrier()
        pltpu.sync_copy(acc.at[pl.ds(sid * TROWS, TROWS)],
                        out.at[cid].at[pl.ds(sid * TROWS, TROWS)])

    return deg_k


def _make_agg(nc):
    @functools.partial(
        pl.kernel,
        out_type=jax.ShapeDtypeStruct((NPAD, 128), F32),
        mesh=_mesh,
        compiler_params=_sc_params,
        scratch_types=[
            pltpu.VMEM((14, 128), jnp.int32),
            pltpu.VMEM((14, 128), jnp.int32),
            pltpu.VMEM((14, 128), jnp.int32),
            pltpu.VMEM((14, 128), jnp.int32),
            pltpu.VMEM((4, 128, 32), F32),
            pltpu.VMEM((TROWS // 128, 128), jnp.int32),
            pltpu.VMEM_SHARED((NPAD, 32), F32),
            pltpu.SemaphoreType.DMA,
            pltpu.SemaphoreType.DMA,
            pltpu.SemaphoreType.DMA,
            pltpu.SemaphoreType.DMA,
            pltpu.SemaphoreType.DMA,
            pltpu.SemaphoreType.DMA,
            pltpu.SemaphoreType.DMA,
            pltpu.SemaphoreType.DMA,
            pltpu.SemaphoreType.DMA,
        ],
    )
    def agg_k(tablev, src4r, dstr, init4, out,
              src_va, dst_va, src_vb, dst_vb, rows_v, init_v, acc,
              sg0, sg1, sg2, sg3, ss0, ss1, ss2, ss3, si0):
        cid = lax.axis_index("c")
        sid = lax.axis_index("s")
        sg = [sg0, sg1, sg2, sg3]
        ss = [ss0, ss1, ss2, ss3]
        for c in range(nc):
            @pl.when(cid == (c % 2))
            def _():
                pltpu.sync_copy(init4.at[c].at[sid], init_v)

                def ibody(z, carry):
                    pltpu.async_copy(
                        tablev.at[init_v.at[z]],
                        rows_v.at[0], sg0).wait()
                    pltpu.sync_copy(
                        rows_v.at[0],
                        acc.at[pl.ds(sid * TROWS + z * 128, 128)])
                    return carry

                lax.fori_loop(0, TROWS // 128, ibody, 0)
                plsc.subcore_barrier()

                pltpu.async_copy(
                    src4r.at[c].at[sid].at[pl.ds(0, 14)], src_va, si0)
                pltpu.async_copy(dstr.at[sid].at[pl.ds(0, 14)], dst_va, si0)

                def ring(g, src_v, dst_v, src_n, dst_n):
                    base = g * 14
                    pltpu.make_async_copy(
                        src4r.at[c].at[sid].at[pl.ds(base, 14)],
                        src_v, si0).wait()
                    pltpu.make_async_copy(
                        dstr.at[sid].at[pl.ds(base, 14)], dst_v, si0).wait()

                    @pl.when(g + 1 < 392 // 14)
                    def _():
                        nb = (g + 1) * 14
                        pltpu.async_copy(
                            src4r.at[c].at[sid].at[pl.ds(nb, 14)],
                            src_n, si0)
                        pltpu.async_copy(
                            dstr.at[sid].at[pl.ds(nb, 14)], dst_n, si0)

                    gath = [None] * 4
                    for w in range(4):
                        gath[w] = pltpu.async_copy(
                            tablev.at[src_v.at[w]], rows_v.at[w], sg[w])
                    tail = []
                    for j in range(14):
                        w = j % 4
                        gath[w].wait()
                        sdesc = pltpu.async_copy(
                            rows_v.at[w], acc.at[dst_v.at[j]], ss[w],
                            add=True)
                        if j + 4 < 14:
                            sdesc.wait()
                            gath[w] = pltpu.async_copy(
                                tablev.at[src_v.at[j + 4]],
                                rows_v.at[w], sg[w])
                        else:
                            tail.append(sdesc)
                    for sdesc in tail:
                        sdesc.wait()

                def body(g2, carry):
                    ring(2 * g2, src_va, dst_va, src_vb, dst_vb)
                    ring(2 * g2 + 1, src_vb, dst_vb, src_va, dst_va)
                    return carry

                lax.fori_loop(0, 392 // 28, body, 0)
                plsc.subcore_barrier()
                pltpu.sync_copy(
                    acc.at[pl.ds(sid * TROWS, TROWS)],
                    out.at[pl.ds(sid * TROWS, TROWS), pl.ds(32 * c, 32)])

    return agg_k


def _make_gather():
    @functools.partial(
        pl.kernel,
        out_type=[
            jax.ShapeDtypeStruct((2, B, 32), F32),
            jax.ShapeDtypeStruct((2, B, 32), F32),
            jax.ShapeDtypeStruct((B, 32), F32),
            jax.ShapeDtypeStruct((B, 32), F32),
        ],
        mesh=_mesh,
        compiler_params=_sc_params,
        scratch_types=[
            pltpu.VMEM((8, 128), jnp.int32),
            pltpu.VMEM((8, 128), jnp.int32),
            pltpu.VMEM((8, 128), jnp.int32),
            pltpu.VMEM((128, 32), F32),
            pltpu.VMEM((128, 32), F32),
            pltpu.VMEM((128, 32), F32),
            pltpu.SemaphoreType.DMA,
        ],
    )
    def gather_k(agg4v, dis32, idxr, idxra, idxrb, gu, gp, du, dp,
                 idx_v, ia_v, ib_v, r0_v, r1_v, rd_v, sem):
        cid = lax.axis_index("c")
        sid = lax.axis_index("s")
        wid = cid * 16 + sid

        def run(out, outd, base0):
            pltpu.sync_copy(idxr.at[wid], idx_v)
            pltpu.sync_copy(idxra.at[wid], ia_v)
            pltpu.sync_copy(idxrb.at[wid], ib_v)
            for j in range(8):
                base = base0 + j * 128
                pltpu.async_copy(agg4v.at[ia_v.at[j]], r0_v, sem).wait()
                pltpu.sync_copy(r0_v, out.at[0].at[pl.ds(base, 128)])
                pltpu.async_copy(agg4v.at[ib_v.at[j]], r1_v, sem).wait()
                pltpu.sync_copy(r1_v, out.at[1].at[pl.ds(base, 128)])
                pltpu.async_copy(dis32.at[idx_v.at[j]], rd_v, sem).wait()
                pltpu.sync_copy(rd_v, outd.at[pl.ds(base, 128)])

        @pl.when(cid == 0)
        def _():
            run(gu, du, sid * 1024)

        @pl.when(cid == 1)
        def _():
            run(gp, dp, sid * 1024)

    return gather_k


_deg_k = _make_deg()
_agg2_k = _make_agg(2)
_agg4_k = _make_agg(4)
_gather_k = _make_gather()


_BLK = 1600
_NBLK = NPAD // _BLK


def _prep_body(deg2_ref, x_ref, dis_ref, dis32_ref, u0_ref):
    deg = deg2_ref[0, :, 0:1] + deg2_ref[1, :, 0:1] + 1.0
    dis = lax.rsqrt(deg)
    dis_ref[...] = dis
    dis32_ref[...] = jnp.broadcast_to(dis, dis32_ref.shape)
    u0_ref[...] = jnp.concatenate(
        [x_ref[...] * dis, jnp.zeros_like(x_ref)], axis=1)


def _tc_prep(deg2, xp):
    return pl.pallas_call(
        _prep_body,
        grid=(_NBLK,),
        in_specs=[
            pl.BlockSpec((2, _BLK, 16), lambda i: (0, i, 0)),
            pl.BlockSpec((_BLK, 64), lambda i: (i, 0)),
        ],
        out_specs=[
            pl.BlockSpec((_BLK, 1), lambda i: (i, 0)),
            pl.BlockSpec((_BLK, 32), lambda i: (i, 0)),
            pl.BlockSpec((_BLK, 128), lambda i: (i, 0)),
        ],
        out_shape=[
            jax.ShapeDtypeStruct((NPAD, 1), F32),
            jax.ShapeDtypeStruct((NPAD, 32), F32),
            jax.ShapeDtypeStruct((NPAD, 128), F32),
        ],
    )(deg2, xp)


def _make_fused(nc_in, nc_out, with_w3):
    din = 32 * nc_in

    def body(agg_ref, dis_ref, w_ref, b_ref, g_ref, bt_ref, w3_ref,
             u_ref, y_ref, st_ref):
        p = pl.program_id(0)
        i = pl.program_id(1)
        dis = dis_ref[...]

        @pl.when(p == 0)
        def _():
            m = agg_ref[:, 0:din] * dis
            y = jnp.dot(m, w_ref[...], preferred_element_type=F32) \
                + b_ref[...]
            y_ref[pl.ds(i * _BLK, _BLK), :] = y

            @pl.when(i == 0)
            def _():
                st_ref[...] = jnp.zeros_like(st_ref)

            rid = i * _BLK + lax.broadcasted_iota(jnp.int32, (_BLK, 1), 0)
            ym = jnp.where(rid < N_NODES, y, 0.0)
            st_ref[0:1] += jnp.sum(ym, axis=0, keepdims=True)
            st_ref[1:2] += jnp.sum(ym * ym, axis=0, keepdims=True)

        @pl.when(p == 1)
        def _():
            mean = st_ref[0:1] / float(N_NODES)
            var = st_ref[1:2] / float(N_NODES) - mean * mean
            inv = lax.rsqrt(var + 1e-5)
            y = y_ref[pl.ds(i * _BLK, _BLK), :]
            h = jnp.maximum((y - mean) * inv * g_ref[...] + bt_ref[...], 0.0)
            if with_w3:
                h = jnp.dot(h, w3_ref[...], preferred_element_type=F32)
            t = h * dis
            if 32 * nc_out < 128:
                t = jnp.concatenate(
                    [t, jnp.zeros((t.shape[0], 128 - 32 * nc_out), F32)],
                    axis=1)
            u_ref[...] = t

    def run(agg, dis, w, b, g, bt, w3):
        dout = w.shape[1]
        return pl.pallas_call(
            body,
            grid=(2, _NBLK),
            in_specs=[
                pl.BlockSpec((_BLK, 128),
                             lambda p, i: (i * (1 - p), 0)),
                pl.BlockSpec((_BLK, 1), lambda p, i: (i, 0)),
                pl.BlockSpec((din, dout), lambda p, i: (0, 0)),
                pl.BlockSpec((1, dout), lambda p, i: (0, 0)),
                pl.BlockSpec((1, dout), lambda p, i: (0, 0)),
                pl.BlockSpec((1, dout), lambda p, i: (0, 0)),
                pl.BlockSpec(w3.shape, lambda p, i: (0, 0)),
            ],
            out_specs=pl.BlockSpec((_BLK, 128), lambda p, i: (i, 0)),
            out_shape=jax.ShapeDtypeStruct((NPAD, 128), F32),
            scratch_shapes=[
                pltpu.VMEM((NPAD, dout), F32),
                pltpu.VMEM((8, dout), F32),
            ],
            compiler_params=pltpu.CompilerParams(
                vmem_limit_bytes=50 * 1024 * 1024),
        )(agg, dis, w, b, g, bt, w3)

    return run


_fused1 = _make_fused(2, 4, False)
_fused2 = _make_fused(4, 2, True)


def _bn_full(a, g, bt):
    m = jnp.mean(a, axis=0, keepdims=True)
    v = jnp.mean(a * a, axis=0, keepdims=True) - m * m
    return jnp.maximum((a - m) * lax.rsqrt(v + 1e-5) * g + bt, 0.0)


_PBLK = 4096
_PNB = B // _PBLK


def _pair_body(gu_ref, gp_ref, du_ref, dp_ref, b3_ref,
               p1_ref, pb1_ref, pg1_ref, pbt1_ref,
               p2_ref, pb2_ref, pg2_ref, pbt2_ref, p3_ref, pb3_ref,
               out_ref, a1_ref, a2_ref, st1_ref, st2_ref):
    p = pl.program_id(0)
    i = pl.program_id(1)
    rows = pl.ds(i * _PBLK, _PBLK)

    @pl.when(p == 0)
    def _():
        ue = jnp.concatenate([gu_ref[0], gu_ref[1]], axis=1)
        pe = jnp.concatenate([gp_ref[0], gp_ref[1]], axis=1)
        p1t = p1_ref[0:64]
        p1b = p1_ref[64:128]
        bias = (jnp.dot(b3_ref[...], p1t + p1b, preferred_element_type=F32)
                + pb1_ref[...])
        a1 = (du_ref[:, 0:1] * jnp.dot(ue, p1t, preferred_element_type=F32)
              + dp_ref[:, 0:1] * jnp.dot(pe, p1b, preferred_element_type=F32)
              + bias)
        a1_ref[rows, :] = a1

        @pl.when(i == 0)
        def _():
            st1_ref[...] = jnp.zeros_like(st1_ref)

        st1_ref[0:1] += jnp.sum(a1, axis=0, keepdims=True)
        st1_ref[1:2] += jnp.sum(a1 * a1, axis=0, keepdims=True)

    @pl.when(p == 1)
    def _():
        mean = st1_ref[0:1] / float(B)
        var = st1_ref[1:2] / float(B) - mean * mean
        z1 = jnp.maximum((a1_ref[rows, :] - mean) * lax.rsqrt(var + 1e-5)
                         * pg1_ref[...] + pbt1_ref[...], 0.0)
        a2 = jnp.dot(z1, p2_ref[...], preferred_element_type=F32) \
            + pb2_ref[...]
        a2_ref[rows, :] = a2

        @pl.when(i == 0)
        def _():
            st2_ref[...] = jnp.zeros_like(st2_ref)

        st2_ref[0:1] += jnp.sum(a2, axis=0, keepdims=True)
        st2_ref[1:2] += jnp.sum(a2 * a2, axis=0, keepdims=True)

    @pl.when(p == 2)
    def _():
        mean = st2_ref[0:1] / float(B)
        var = st2_ref[1:2] / float(B) - mean * mean
        z2 = jnp.maximum((a2_ref[rows, :] - mean) * lax.rsqrt(var + 1e-5)
                         * pg2_ref[...] + pbt2_ref[...], 0.0)
        a3 = jnp.dot(z2, p3_ref[...], preferred_element_type=F32) \
            + pb3_ref[...]
        out_ref[...] = jax.nn.sigmoid(a3) * 4.0 + 1.0


def _tc_pair(gu, gp, du, dp, b3, p1, pb1, pg1, pbt1,
             p2, pb2, pg2, pbt2, p3, pb3):
    full = lambda shape: pl.BlockSpec(shape, lambda p, i: tuple(
        0 for _ in shape))
    return pl.pallas_call(
        _pair_body,
        grid=(3, _PNB),
        in_specs=[
            pl.BlockSpec((2, _PBLK, 32), lambda p, i: (0, jnp.where(p == 0, i, 0), 0)),
            pl.BlockSpec((2, _PBLK, 32), lambda p, i: (0, jnp.where(p == 0, i, 0), 0)),
            pl.BlockSpec((_PBLK, 32), lambda p, i: (jnp.where(p == 0, i, 0), 0)),
            pl.BlockSpec((_PBLK, 32), lambda p, i: (jnp.where(p == 0, i, 0), 0)),
            full((1, 64)),
            full((128, 128)),
            full((1, 128)),
            full((1, 128)),
            full((1, 128)),
            full((128, 64)),
            full((1, 64)),
            full((1, 64)),
            full((1, 64)),
            full((64, 1)),
            full((1, 1)),
        ],
        out_specs=pl.BlockSpec((_PBLK, 1), lambda p, i: (i, 0)),
        out_shape=jax.ShapeDtypeStruct((B, 1), F32),
        scratch_shapes=[
            pltpu.VMEM((B, 128), F32),
            pltpu.VMEM((B, 64), F32),
            pltpu.VMEM((8, 128), F32),
            pltpu.VMEM((8, 64), F32),
        ],
        compiler_params=pltpu.CompilerParams(
            vmem_limit_bytes=50 * 1024 * 1024),
    )(gu, gp, du, dp, b3, p1, pb1, pg1, pbt1, p2, pb2, pg2, pbt2, p3, pb3)


def kernel(x, edge_index, user_indices, product_indices,
           W1, b1, g1, bt1, W2, b2, g2, bt2, W3, b3,
           P1, pb1, pg1, pbt1, P2, pb2, pg2, pbt2, P3, pb3):
    src = jnp.concatenate(
        [edge_index[0], jnp.zeros((EPAD - N_EDGES,), jnp.int32)])
    dst = jnp.concatenate(
        [edge_index[1], jnp.full((EPAD - N_EDGES,), DUMMY, jnp.int32)])
    src4r = ((src * 4)[None, :]
             + jnp.arange(4, dtype=jnp.int32)[:, None]).reshape(
                 4, 16, 392, 128)
    init4 = ((jnp.arange(NPAD, dtype=jnp.int32) * 4)[None, :]
             + jnp.arange(4, dtype=jnp.int32)[:, None]).reshape(
                 4, 16, TROWS // 128, 128)
    dstr16 = dst.reshape(16, 392, 128)
    dstr32 = dst.reshape(32, 196, 128)
    xp = jnp.pad(x, ((0, NPAD - N_NODES), (0, 0)))
    idx_all = jnp.concatenate([user_indices + USER_OFFSET, product_indices])
    idxr = idx_all.reshape(32, 8, 128)
    idxra = (idx_all * 4).reshape(32, 8, 128)
    idxrb = (idx_all * 4 + 1).reshape(32, 8, 128)
    ones16 = jnp.ones((128, 16), F32)
    zeros16 = jnp.zeros((128, 16), F32)
    b1r, b2r, b3r = b1.reshape(1, -1), b2.reshape(1, -1), b3.reshape(1, -1)
    g1r, g2r = g1.reshape(1, -1), g2.reshape(1, -1)
    bt1r, bt2r = bt1.reshape(1, -1), bt2.reshape(1, -1)
    pb1r, pb2r, pb3r = pb1.reshape(1, -1), pb2.reshape(1, -1), pb3.reshape(1, -1)
    pg1r, pg2r = pg1.reshape(1, -1), pg2.reshape(1, -1)
    pbt1r, pbt2r = pbt1.reshape(1, -1), pbt2.reshape(1, -1)

    deg2 = _deg_k(dstr32, ones16, zeros16)
    dis, dis32, u0 = _tc_prep(deg2, xp)

    agg0 = _agg2_k(u0.reshape(NPAD * 4, 32), src4r, dstr16, init4)
    u1 = _fused1(agg0, dis, W1, b1r, g1r, bt1r, jnp.zeros((1, 1), F32))

    agg1 = _agg4_k(u1.reshape(NPAD * 4, 32), src4r, dstr16, init4)
    u2 = _fused2(agg1, dis, W2, b2r, g2r, bt2r, W3)

    agg2 = _agg2_k(u2.reshape(NPAD * 4, 32), src4r, dstr16, init4)

    gu, gp, du, dp = _gather_k(agg2.reshape(NPAD * 4, 32), dis32,
                               idxr, idxra, idxrb)
    out = _tc_pair(gu, gp, du, dp, b3r, P1, pb1r, pg1r, pbt1r,
                   P2, pb2r, pg2r, pbt2r, P3, pb3r)
    return out.reshape(B)
